# baseline (speedup 1.0000x reference)
import functools

import jax
import jax.numpy as jnp
from jax import lax
from jax.experimental import pallas as pl
from jax.experimental.pallas import tpu as pltpu
from jax.experimental.pallas import tpu_sc as plsc

NC = 2
NS = 16
NW = NC * NS

LANES = 16
NBUF = 8
GLAG = 6


def _make_gather(n_tok: int, n_seq: int, D: int):
    assert n_tok % 128 == 0 and D % 8 == 0
    n_blocks = n_seq * (n_tok // 128)
    assert n_blocks % NW == 0
    nblk = n_blocks // NW
    out_rows = n_blocks * D
    n_iter = -(-(nblk + GLAG) // NBUF) * NBUF
    mesh = plsc.VectorSubcoreMesh(core_axis_name="c", subcore_axis_name="s")

    @functools.partial(
        pl.kernel,
        out_type=jax.ShapeDtypeStruct((out_rows, 128), jnp.float32),
        mesh=mesh,
        scratch_types=[
            pltpu.VMEM((nblk * 128,), jnp.int32),
            [pltpu.VMEM((128, D), jnp.float32) for _ in range(NBUF)],
            [pltpu.VMEM((D, 128), jnp.float32) for _ in range(2)],
            [pltpu.SemaphoreType.DMA for _ in range(NBUF)],
            [pltpu.SemaphoreType.DMA for _ in range(2)],
        ],
        compiler_params=pltpu.CompilerParams(
            use_tc_tiling_on_sc=False, needs_layout_passes=False,
            disable_bounds_checks=True),
    )
    def gather_kernel(xt_hbm, w_hbm, out_hbm,
                      idx_all, rows, outb, gsem, wsem):
        wid = lax.axis_index("s") * NC + lax.axis_index("c")
        blk0 = wid * nblk

        def fire_gather(g, b):
            pltpu.async_copy(
                w_hbm.at[idx_all.at[pl.ds(pl.multiple_of(g * 128, 128), 128)]],
                rows[b], gsem[b])

        def wait_gather(b):
            pltpu.make_async_copy(
                w_hbm.at[pl.ds(0, 128)], rows[b], gsem[b]).wait()

        def transpose(rb, ob):
            @plsc.parallel_loop(0, D, unroll=4)
            def _t(d):
                for c in range(128 // LANES):
                    v = plsc.load_gather(
                        rows[rb],
                        [lax.iota(jnp.int32, LANES) + (c * LANES),
                         jnp.full((LANES,), d, jnp.int32)],
                    )
                    outb[ob][d, pl.ds(c * LANES, LANES)] = v

        def fire_writes(k, ob):
            B = blk0 + k
            base = pl.multiple_of(((B >> 7) << 12) + ((B & 127) << 3), 8)
            for tr in range(D // 8):
                pltpu.async_copy(
                    outb[ob].at[pl.ds(tr * 8, 8)],
                    out_hbm.at[pl.ds(base + tr * 1024, 8)],
                    wsem[ob],
                )

        def wait_writes(ob):
            pltpu.make_async_copy(
                outb[ob], out_hbm.at[pl.ds(0, D)], wsem[ob]).wait()

        pltpu.sync_copy(xt_hbm.at[pl.ds(blk0 * 128, nblk * 128)], idx_all)

        @pl.loop(0, n_iter, step=NBUF)
        def _step(g0):
            for b in range(NBUF):
                g = g0 + b

                @pl.when(g < nblk)
                def _():
                    fire_gather(g, b)

                k = g - GLAG
                kb = (b + NBUF - GLAG) % NBUF

                @pl.when((k >= 0) & (k < nblk))
                def _():
                    wait_gather(kb)

                    @pl.when(k >= 2)
                    def _():
                        wait_writes((b + NBUF - GLAG) % 2)

                    transpose(kb, (b + NBUF - GLAG) % 2)
                    fire_writes(k, (b + NBUF - GLAG) % 2)

        wait_writes(0)
        wait_writes(1)

    return gather_kernel


def _tc_transpose(wt):
    D, V = wt.shape
    CB = 512
    grid = -(-V // CB)

    def body(in_ref, out_ref):
        out_ref[...] = in_ref[...].T

    return pl.pallas_call(
        body,
        grid=(grid,),
        in_specs=[pl.BlockSpec((D, CB), lambda j: (0, j))],
        out_specs=pl.BlockSpec((CB, D), lambda j: (j, 0)),
        out_shape=jax.ShapeDtypeStruct((V, D), jnp.float32),
    )(wt)


def kernel(x, weight):
    n_tok, n_seq = x.shape
    D = weight.shape[1]
    xt = x.T.reshape(-1).astype(jnp.int32)
    w_lin = _tc_transpose(weight.T)
    out_k = _make_gather(n_tok, n_seq, D)(xt, w_lin)
    out = (
        out_k.reshape(n_seq, D // 8, n_tok // 128, 8, 128)
        .transpose(2, 4, 0, 1, 3)
        .reshape(n_tok, n_seq, D)
    )
    return out


# speedup vs baseline: 2.1444x; 2.1444x over previous
import functools

import jax
import jax.numpy as jnp
from jax import lax
from jax.experimental import pallas as pl
from jax.experimental.pallas import tpu as pltpu
from jax.experimental.pallas import tpu_sc as plsc

NC = 2
NS = 16
NW = NC * NS

LANES = 16
NBUF = 8
GLAG = 6


def _make_gather(n_tok: int, n_seq: int, D: int):
    assert n_tok % 128 == 0 and D % 8 == 0
    n_blocks = n_seq * (n_tok // 128)
    assert n_blocks % NW == 0
    nblk = n_blocks // NW
    out_rows = n_blocks * D
    n_iter = -(-(nblk + GLAG) // NBUF) * NBUF
    mesh = plsc.VectorSubcoreMesh(core_axis_name="c", subcore_axis_name="s")

    @functools.partial(
        pl.kernel,
        out_type=jax.ShapeDtypeStruct((out_rows, 128), jnp.float32),
        mesh=mesh,
        scratch_types=[
            pltpu.VMEM((nblk * 128,), jnp.int32),
            [pltpu.VMEM((128, D), jnp.float32) for _ in range(NBUF)],
            [pltpu.VMEM((D, 128), jnp.float32) for _ in range(2)],
            [pltpu.SemaphoreType.DMA for _ in range(NBUF)],
            [pltpu.SemaphoreType.DMA for _ in range(2)],
        ],
        compiler_params=pltpu.CompilerParams(
            use_tc_tiling_on_sc=False, needs_layout_passes=False,
            disable_bounds_checks=True),
    )
    def gather_kernel(xt_hbm, w_hbm, out_hbm,
                      idx_all, rows, outb, gsem, wsem):
        wid = lax.axis_index("s") * NC + lax.axis_index("c")
        blk0 = wid * nblk

        def fire_gather(g, b):
            pltpu.async_copy(
                w_hbm.at[idx_all.at[pl.ds(pl.multiple_of(g * 128, 128), 128)]],
                rows[b], gsem[b])

        def wait_gather(b):
            pltpu.make_async_copy(
                w_hbm.at[pl.ds(0, 128)], rows[b], gsem[b]).wait()

        def transpose(rb, ob):
            @plsc.parallel_loop(0, 128, unroll=8)
            def _t(t):
                for h in range(D // LANES):
                    v = rows[rb][t, pl.ds(h * LANES, LANES)]
                    plsc.store_scatter(
                        outb[ob],
                        [lax.iota(jnp.int32, LANES) + (h * LANES),
                         jnp.full((LANES,), t, jnp.int32)],
                        v,
                    )

        def fire_writes(k, ob):
            B = blk0 + k
            base = pl.multiple_of(((B >> 7) << 12) + ((B & 127) << 3), 8)
            for tr in range(D // 8):
                pltpu.async_copy(
                    outb[ob].at[pl.ds(tr * 8, 8)],
                    out_hbm.at[pl.ds(base + tr * 1024, 8)],
                    wsem[ob],
                )

        def wait_writes(ob):
            pltpu.make_async_copy(
                outb[ob], out_hbm.at[pl.ds(0, D)], wsem[ob]).wait()

        pltpu.sync_copy(xt_hbm.at[pl.ds(blk0 * 128, nblk * 128)], idx_all)

        @pl.loop(0, n_iter, step=NBUF)
        def _step(g0):
            for b in range(NBUF):
                g = g0 + b

                @pl.when(g < nblk)
                def _():
                    fire_gather(g, b)

                k = g - GLAG
                kb = (b + NBUF - GLAG) % NBUF

                @pl.when((k >= 0) & (k < nblk))
                def _():
                    wait_gather(kb)

                    @pl.when(k >= 2)
                    def _():
                        wait_writes((b + NBUF - GLAG) % 2)

                    transpose(kb, (b + NBUF - GLAG) % 2)
                    fire_writes(k, (b + NBUF - GLAG) % 2)

        wait_writes(0)
        wait_writes(1)

    return gather_kernel


def kernel(x, weight):
    n_tok, n_seq = x.shape
    D = weight.shape[1]
    xt = x.T.reshape(-1).astype(jnp.int32)
    out_k = _make_gather(n_tok, n_seq, D)(xt, weight)
    out = (
        out_k.reshape(n_seq, D // 8, n_tok // 128, 8, 128)
        .transpose(2, 4, 0, 1, 3)
        .reshape(n_tok, n_seq, D)
    )
    return out
